# Initial kernel scaffold; baseline (speedup 1.0000x reference)
import jax
import jax.numpy as jnp
from jax.experimental import pallas as pl


def kernel(integer_tokens, token_embedding):
    raise NotImplementedError("write your pallas kernel here")


# speedup vs baseline: 3.7230x; 3.7230x over previous
import functools

import jax
import jax.numpy as jnp
from jax import lax
from jax.experimental import pallas as pl
from jax.experimental.pallas import tpu as pltpu
from jax.experimental.pallas import tpu_sc as plsc

_NC = 2
_NS = 16
_NW = _NC * _NS
_CHUNK = 128


@functools.partial(jax.jit, static_argnums=(2, 3))
def _sc_embedding_gather(tokens_flat, table, b, d):
    b_per_w = b // _NW
    n_chunks = b_per_w // _CHUNK
    mesh = plsc.VectorSubcoreMesh(core_axis_name="c", subcore_axis_name="s")

    @functools.partial(
        pl.kernel,
        mesh=mesh,
        out_type=jax.ShapeDtypeStruct((b, d), jnp.float32),
        scratch_types=[
            pltpu.VMEM((_CHUNK,), jnp.int32),
            pltpu.VMEM((_CHUNK, d), jnp.float32),
            pltpu.SemaphoreType.DMA,
        ],
    )
    def k(tok_hbm, tab_hbm, out_hbm, idx_v, rows_v, sem):
        wid = lax.axis_index("s") * _NC + lax.axis_index("c")
        base = wid * b_per_w

        def body(i, carry):
            off = base + i * _CHUNK
            pltpu.sync_copy(tok_hbm.at[pl.ds(off, _CHUNK)], idx_v)
            pltpu.async_copy(tab_hbm.at[idx_v], rows_v, sem).wait()
            pltpu.sync_copy(rows_v, out_hbm.at[pl.ds(off, _CHUNK)])
            return carry

        lax.fori_loop(0, n_chunks, body, 0)

    return k(tokens_flat, table)


def kernel(integer_tokens, token_embedding):
    bsz, seq = integer_tokens.shape
    d = token_embedding.shape[1]
    flat = integer_tokens.reshape(bsz * seq)
    out = _sc_embedding_gather(flat, token_embedding, bsz * seq, d)
    return out.reshape(bsz, seq, d)


# speedup vs baseline: 4.2083x; 1.1304x over previous
import functools

import jax
import jax.numpy as jnp
from jax import lax
from jax.experimental import pallas as pl
from jax.experimental.pallas import tpu as pltpu
from jax.experimental.pallas import tpu_sc as plsc

_NC = 2
_NS = 16
_NW = _NC * _NS
_CHUNK = 128


@functools.partial(jax.jit, static_argnums=(2, 3))
def _sc_embedding_gather(tokens_2d, table, b, d):
    b_per_w = b // _NW
    n_chunks = b_per_w // _CHUNK
    mesh = plsc.VectorSubcoreMesh(core_axis_name="c", subcore_axis_name="s")

    @functools.partial(
        pl.kernel,
        mesh=mesh,
        out_type=jax.ShapeDtypeStruct((b, d), jnp.float32),
        scratch_types=[
            pltpu.VMEM((n_chunks, _CHUNK), jnp.int32),
            pltpu.VMEM((_CHUNK, d), jnp.float32),
            pltpu.VMEM((_CHUNK, d), jnp.float32),
            pltpu.SemaphoreType.DMA,
            pltpu.SemaphoreType.DMA,
        ],
    )
    def k(tok_hbm, tab_hbm, out_hbm, idx_v, rows_a, rows_b, sem_a, sem_b):
        wid = lax.axis_index("s") * _NC + lax.axis_index("c")
        base = wid * b_per_w

        pltpu.sync_copy(tok_hbm.at[wid], idx_v)

        def gather_start(c, buf, sem):
            pltpu.make_async_copy(tab_hbm.at[idx_v.at[c]], buf, sem).start()

        def gather_wait(buf, sem):
            pltpu.make_async_copy(tab_hbm.at[idx_v.at[0]], buf, sem).wait()

        def write_out(c, buf):
            pltpu.sync_copy(buf, out_hbm.at[pl.ds(base + c * _CHUNK, _CHUNK)])

        gather_start(0, rows_a, sem_a)

        def body(j, carry):
            c0 = 2 * j
            c1 = c0 + 1
            gather_wait(rows_a, sem_a)
            gather_start(c1, rows_b, sem_b)
            write_out(c0, rows_a)
            gather_wait(rows_b, sem_b)
            gather_start(lax.min(c0 + 2, n_chunks - 1), rows_a, sem_a)
            write_out(c1, rows_b)
            return carry

        lax.fori_loop(0, n_chunks // 2, body, 0)
        gather_wait(rows_a, sem_a)

    return k(tokens_2d, table)


def kernel(integer_tokens, token_embedding):
    bsz, seq = integer_tokens.shape
    d = token_embedding.shape[1]
    tok3d = integer_tokens.reshape(_NW, bsz * seq // (_NW * _CHUNK), _CHUNK)
    out = _sc_embedding_gather(tok3d, token_embedding, bsz * seq, d)
    return out.reshape(bsz, seq, d)
